# baseline (speedup 1.0000x reference)
import functools

import jax
import jax.numpy as jnp
from jax.experimental import pallas as pl
from jax.experimental.pallas import tpu as pltpu

M = 1024
K = 16
N = 100000
TILE_R = 1568
NSLOTS = 4
NT = (N + TILE_R - 1) // TILE_R
LAST_R = N - (NT - 1) * TILE_R


def _mm_kernel(em_ref, x_ref, o_ref, acc_ref, tail_ref, sem_ref, tail_sem):
    i = pl.program_id(0)
    slot = jax.lax.rem(i, NSLOTS)

    @pl.when(jnp.logical_and(i >= NSLOTS, i < NT - 1))
    def _wait_prev():
        pltpu.make_async_copy(
            acc_ref.at[slot],
            o_ref.at[pl.ds((i - NSLOTS) * TILE_R, TILE_R), :],
            sem_ref.at[slot],
        ).wait()

    @pl.when(i < NT - 1)
    def _store_full():
        acc_ref[slot] = jax.lax.dot_general(
            em_ref[...], x_ref[...],
            dimension_numbers=(((1,), (1,)), ((), ())),
            preferred_element_type=jnp.float32,
        )
        pltpu.make_async_copy(
            acc_ref.at[slot],
            o_ref.at[pl.ds(i * TILE_R, TILE_R), :],
            sem_ref.at[slot],
        ).start()

    @pl.when(i == NT - 1)
    def _store_last_and_drain():
        tail_ref[...] = jax.lax.dot_general(
            em_ref[:LAST_R, :], x_ref[...],
            dimension_numbers=(((1,), (1,)), ((), ())),
            preferred_element_type=jnp.float32,
        )
        last = pltpu.make_async_copy(
            tail_ref,
            o_ref.at[pl.ds((NT - 1) * TILE_R, LAST_R), :],
            tail_sem,
        )
        last.start()
        for back in range(1, NSLOTS + 1):
            j = NT - 1 - back
            if j >= 0:
                pltpu.make_async_copy(
                    acc_ref.at[j % NSLOTS],
                    o_ref.at[pl.ds(j * TILE_R, TILE_R), :],
                    sem_ref.at[j % NSLOTS],
                ).wait()
        last.wait()


@functools.partial(jax.jit, static_argnames=())
def kernel(inputs, targets, em):
    del targets
    out_t = pl.pallas_call(
        _mm_kernel,
        grid=(NT,),
        in_specs=[
            pl.BlockSpec((TILE_R, K), lambda i: (i, 0)),
            pl.BlockSpec((M, K), lambda i: (0, 0)),
        ],
        out_specs=pl.BlockSpec(memory_space=pl.ANY),
        out_shape=jax.ShapeDtypeStruct((N, M), jnp.float32),
        scratch_shapes=[
            pltpu.VMEM((NSLOTS, TILE_R, M), jnp.float32),
            pltpu.VMEM((LAST_R, M), jnp.float32),
            pltpu.SemaphoreType.DMA((NSLOTS,)),
            pltpu.SemaphoreType.DMA,
        ],
        compiler_params=pltpu.CompilerParams(
            dimension_semantics=("arbitrary",),
        ),
    )(em, inputs)
    return out_t.T


# speedup vs baseline: 1.0217x; 1.0217x over previous
import functools

import jax
import jax.numpy as jnp
from jax.experimental import pallas as pl
from jax.experimental.pallas import tpu as pltpu

TILE_R = 6272


def _mm_kernel(em_ref, x_ref, o_ref):
    o_ref[...] = jax.lax.dot_general(
        em_ref[...], x_ref[...],
        dimension_numbers=(((1,), (1,)), ((), ())),
        preferred_element_type=jnp.float32,
    )


@functools.partial(jax.jit, static_argnames=())
def kernel(inputs, targets, em):
    del targets
    m, k = inputs.shape
    n = em.shape[0]
    out_t = pl.pallas_call(
        _mm_kernel,
        grid=(pl.cdiv(n, TILE_R),),
        in_specs=[
            pl.BlockSpec((TILE_R, k), lambda i: (i, 0)),
            pl.BlockSpec((m, k), lambda i: (0, 0)),
        ],
        out_specs=pl.BlockSpec((TILE_R, m), lambda i: (i, 0)),
        out_shape=jax.ShapeDtypeStruct((n, m), jnp.float32),
        compiler_params=pltpu.CompilerParams(
            dimension_semantics=("arbitrary",),
        ),
    )(em, inputs)
    return out_t.T


# speedup vs baseline: 1.0223x; 1.0006x over previous
import functools

import jax
import jax.numpy as jnp
from jax.experimental import pallas as pl
from jax.experimental.pallas import tpu as pltpu

TILE_R = 6272


def _mm_kernel(em_ref, x_ref, o_ref):
    o_ref[...] = jax.lax.dot_general(
        em_ref[...].astype(jnp.bfloat16), x_ref[...].astype(jnp.bfloat16),
        dimension_numbers=(((1,), (1,)), ((), ())),
        preferred_element_type=jnp.float32,
    )


@functools.partial(jax.jit, static_argnames=())
def kernel(inputs, targets, em):
    del targets
    m, k = inputs.shape
    n = em.shape[0]
    out_t = pl.pallas_call(
        _mm_kernel,
        grid=(pl.cdiv(n, TILE_R),),
        in_specs=[
            pl.BlockSpec((TILE_R, k), lambda i: (i, 0)),
            pl.BlockSpec((m, k), lambda i: (0, 0)),
        ],
        out_specs=pl.BlockSpec((TILE_R, m), lambda i: (i, 0)),
        out_shape=jax.ShapeDtypeStruct((n, m), jnp.float32),
        compiler_params=pltpu.CompilerParams(
            dimension_semantics=("arbitrary",),
        ),
    )(em, inputs)
    return out_t.T
